# Initial kernel scaffold; baseline (speedup 1.0000x reference)
import jax
import jax.numpy as jnp
from jax.experimental import pallas as pl


def kernel(x, edge_index, W1, b1, W2, b2):
    raise NotImplementedError("write your pallas kernel here")


# speedup vs baseline: 11.7427x; 11.7427x over previous
import functools

import jax
import jax.numpy as jnp
from jax import lax
from jax.experimental import pallas as pl
from jax.experimental.pallas import tpu as pltpu
from jax.experimental.pallas import tpu_sc as plsc

N = 10000
E = 160000
D = 256
H = D // 2
NC = 2
NS = 16
L = 16

K = 80
CH = E // NS // K

DEG_TILE = 64 * K
E_PAD = 32 * DEG_TILE
NPAD = 10112
PROWS = NPAD // NS

@functools.cache
def _mesh():
    return plsc.VectorSubcoreMesh(
        core_axis_name="c", subcore_axis_name="s", num_cores=NC, num_subcores=NS
    )


def _sc_degree_body(dst_hbm, out_hbm, idx_v, ones_v, zero_v, acc):
    c = lax.axis_index("c")
    s = lax.axis_index("s")
    wid = s * NC + c

    def fill_ones(i, _):
        ones_v[i, :] = jnp.full((L,), 1.0, jnp.float32)
        return 0

    lax.fori_loop(0, K, fill_ones, 0)

    def fill_zero(i, _):
        zero_v[i, :] = jnp.zeros((L,), jnp.float32)
        return 0

    lax.fori_loop(0, 8, fill_zero, 0)

    def zero_acc(i, _):
        pltpu.sync_copy(zero_v, acc.at[pl.ds(s * PROWS + i * 8, 8)])
        return 0

    lax.fori_loop(0, PROWS // 8, zero_acc, 0)
    plsc.subcore_barrier()

    pltpu.sync_copy(dst_hbm.at[wid], idx_v)

    def body(j, _):
        pltpu.sync_copy(ones_v, acc.at[idx_v.at[j]], add=True)
        return 0

    lax.fori_loop(0, 64, body, 0)
    plsc.subcore_barrier()
    pltpu.sync_copy(
        acc.at[pl.ds(s * PROWS, PROWS)],
        out_hbm.at[pl.ds(c * NPAD + s * PROWS, PROWS)],
    )


@functools.cache
def _sc_degree():
    return pl.kernel(
        _sc_degree_body,
        out_type=jax.ShapeDtypeStruct((NC * NPAD, L), jnp.float32),
        mesh=_mesh(),
        scratch_types=[
            pltpu.VMEM((64, K), jnp.int32),
            pltpu.VMEM((K, L), jnp.float32),
            pltpu.VMEM((8, L), jnp.float32),
            pltpu.VMEM_SHARED((NPAD, L), jnp.float32),
        ],
    )


def _sc_aggregate_body(z_hbm, src_hbm, dst_hbm, out_hbm, src_v, dst_v, rows_v, zero_v, acc):
    c = lax.axis_index("c")
    s = lax.axis_index("s")

    def fill_zero(i, _):
        for q in range(H // L):
            zero_v[i, pl.ds(q * L, L)] = jnp.zeros((L,), jnp.float32)
        return 0

    lax.fori_loop(0, 8, fill_zero, 0)

    def zero_acc(i, _):
        pltpu.sync_copy(zero_v, acc.at[pl.ds(s * PROWS + i * 8, 8)])
        return 0

    lax.fori_loop(0, PROWS // 8, zero_acc, 0)

    pltpu.sync_copy(src_hbm.at[s], src_v)
    pltpu.sync_copy(dst_hbm.at[s], dst_v)
    off = jnp.full((L,), c * N, jnp.int32)

    def add_off(j, _):
        for q in range(K // L):
            src_v[j, pl.ds(q * L, L)] = src_v[j, pl.ds(q * L, L)] + off
        return 0

    lax.fori_loop(0, CH, add_off, 0)
    plsc.subcore_barrier()

    def body(j, _):
        pltpu.sync_copy(z_hbm.at[src_v.at[j]], rows_v)
        pltpu.sync_copy(rows_v, acc.at[dst_v.at[j]], add=True)
        return 0

    lax.fori_loop(0, CH, body, 0)
    plsc.subcore_barrier()
    pltpu.sync_copy(
        acc.at[pl.ds(s * PROWS, PROWS)],
        out_hbm.at[pl.ds(c * NPAD + s * PROWS, PROWS)],
    )


@functools.cache
def _sc_aggregate():
    return pl.kernel(
        _sc_aggregate_body,
        out_type=jax.ShapeDtypeStruct((NC * NPAD, H), jnp.float32),
        mesh=_mesh(),
        scratch_types=[
            pltpu.VMEM((CH, K), jnp.int32),
            pltpu.VMEM((CH, K), jnp.int32),
            pltpu.VMEM((K, H), jnp.float32),
            pltpu.VMEM((8, H), jnp.float32),
            pltpu.VMEM_SHARED((NPAD, H), jnp.float32),
        ],
    )


R = 1000


def _dinv_of(degp):
    deg = degp[0, :, :1] + degp[1, :, :1] + 1.0
    return lax.rsqrt(deg)


def _tc_first_body(x_ref, w_ref, degp_ref, z_ref):
    dinv = _dinv_of(degp_ref[...])
    xw = jnp.dot(x_ref[...], w_ref[...], preferred_element_type=jnp.float32)
    z = xw * dinv
    z_ref[0] = z[:, :H]
    z_ref[1] = z[:, H:]


def _tc_mid_body(s_ref, z_ref, degp_ref, b_ref, w_ref, out_ref):
    dinv = _dinv_of(degp_ref[...])
    t = s_ref[...] + z_ref[...]
    h = jnp.concatenate([t[0], t[1]], axis=1) * dinv + b_ref[...]
    h = jnp.maximum(h, 0.0)
    y = jnp.dot(h, w_ref[...], preferred_element_type=jnp.float32) * dinv
    out_ref[0] = y[:, :H]
    out_ref[1] = y[:, H:]


def _tc_out_body(s_ref, z_ref, degp_ref, b_ref, out_ref):
    dinv = _dinv_of(degp_ref[...])
    t = s_ref[...] + z_ref[...]
    out_ref[...] = jnp.concatenate([t[0], t[1]], axis=1) * dinv + b_ref[...]


_spec_rows = pl.BlockSpec((R, D), lambda i: (i, 0))
_spec_w = pl.BlockSpec((D, D), lambda i: (0, 0))
_spec_b = pl.BlockSpec((1, D), lambda i: (0, 0))
_spec_degp = pl.BlockSpec((2, R, L), lambda i: (0, i, 0))
_spec_half = pl.BlockSpec((2, R, H), lambda i: (0, i, 0))


def _tc_first(x, W1, degp):
    return pl.pallas_call(
        _tc_first_body,
        grid=(N // R,),
        in_specs=[_spec_rows, _spec_w, _spec_degp],
        out_specs=_spec_half,
        out_shape=jax.ShapeDtypeStruct((2, N, H), jnp.float32),
    )(x, W1, degp)


def _tc_mid(S, z, degp, b, W2):
    return pl.pallas_call(
        _tc_mid_body,
        grid=(N // R,),
        in_specs=[_spec_half, _spec_half, _spec_degp, _spec_b, _spec_w],
        out_specs=_spec_half,
        out_shape=jax.ShapeDtypeStruct((2, N, H), jnp.float32),
    )(S, z, degp, b, W2)


def _tc_out(S, z, degp, b):
    return pl.pallas_call(
        _tc_out_body,
        grid=(N // R,),
        in_specs=[_spec_half, _spec_half, _spec_degp, _spec_b],
        out_specs=_spec_rows,
        out_shape=jax.ShapeDtypeStruct((N, D), jnp.float32),
    )(S, z, degp, b)


def kernel(x, edge_index, W1, b1, W2, b2):
    src3 = edge_index[0].reshape(NS, CH, K)
    dst3 = edge_index[1].reshape(NS, CH, K)
    dstpad = jnp.concatenate(
        [edge_index[1], jnp.full((E_PAD - E,), N, jnp.int32)]
    ).reshape(32, 64, K)

    deg_raw = _sc_degree()(dstpad)
    degp = deg_raw.reshape(NC, NPAD, L)

    b1r = b1.reshape(1, D)
    b2r = b2.reshape(1, D)

    z1 = _tc_first(x, W1, degp)
    S1 = _sc_aggregate()(z1.reshape(NC * N, H), src3, dst3)
    z2 = _tc_mid(S1.reshape(NC, NPAD, H), z1, degp, b1r, W2)
    S2 = _sc_aggregate()(z2.reshape(NC * N, H), src3, dst3)
    return _tc_out(S2.reshape(NC, NPAD, H), z2, degp, b2r)
